# baseline (speedup 1.0000x reference)
import functools

import jax
import jax.numpy as jnp
from jax import lax
from jax.experimental import pallas as pl
from jax.experimental.pallas import tpu as pltpu
from jax.experimental.pallas import tpu_sc as plsc

N = 100000
E = 1600000
D = 32
HD = D // 2
EPS = 1e-5

NC = 2
NS = 16
NW = NC * NS

TBL = 100096
RPT = TBL // NS
WB_LAST = N - (NS - 1) * RPT
ZR = 782
NZ = RPT // ZR

CHUNK = 128
CPB = 8
NBUF = 4
EROWS = 12800
PAD_E = EROWS * CHUNK - E
ROWS_PT = EROWS // NS
NBLK = ROWS_PT // CPB
ROWS_PW = EROWS // NW
DBLK = ROWS_PW // CPB

_MESH = plsc.VectorSubcoreMesh(
    core_axis_name="c", subcore_axis_name="s", num_cores=NC, num_subcores=NS)
_SC_PARAMS = pltpu.CompilerParams(use_tc_tiling_on_sc=False)


def _zero_table(agg, zv, z_h, s):
  pltpu.sync_copy(z_h, zv)
  for t in range(NZ):
    pltpu.sync_copy(zv, agg.at[pl.ds(s * RPT + t * ZR, ZR)])


def _writeback(agg, out_h, s):
  @pl.when(s < NS - 1)
  def _():
    pltpu.sync_copy(agg.at[pl.ds(s * RPT, RPT)],
                    out_h.at[pl.ds(s * RPT, RPT)])

  @pl.when(s == NS - 1)
  def _():
    pltpu.sync_copy(agg.at[pl.ds((NS - 1) * RPT, WB_LAST)],
                    out_h.at[pl.ds((NS - 1) * RPT, WB_LAST)])


@functools.partial(
    pl.kernel,
    out_type=(jax.ShapeDtypeStruct((N, 8), jnp.float32),
              jax.ShapeDtypeStruct((N, 8), jnp.float32)),
    mesh=_MESH,
    compiler_params=_SC_PARAMS,
    scratch_types=[
        pltpu.VMEM_SHARED((TBL, 8), jnp.float32),
        pltpu.VMEM((RPT, 8), jnp.float32),
        pltpu.VMEM((CHUNK, 8), jnp.float32),
        pltpu.VMEM((CPB, CHUNK), jnp.int32),
        pltpu.SemaphoreType.DMA,
    ])
def _sc_degree(dst2_h, z_h, ones_h, d0_h, d1_h, agg, zv, ov, draw, sem):
  c = lax.axis_index("c")
  s = lax.axis_index("s")
  pltpu.sync_copy(ones_h, ov)
  pltpu.sync_copy(z_h, zv)
  pltpu.sync_copy(zv, agg.at[pl.ds(s * RPT, RPT)])
  plsc.subcore_barrier()

  wid = c * NS + s

  def blk(b, carry):
    roff = wid * ROWS_PW + b * CPB
    pltpu.sync_copy(dst2_h.at[pl.ds(roff, CPB)], draw)
    descs = [pltpu.async_copy(ov, agg.at[draw.at[k]], sem, add=True)
             for k in range(CPB)]
    for d in descs:
      d.wait()
    return carry

  lax.fori_loop(0, DBLK, blk, 0)
  plsc.subcore_barrier()

  @pl.when(c == 0)
  def _():
    _writeback(agg, d0_h, s)

  @pl.when(c == 1)
  def _():
    _writeback(agg, d1_h, s)


@functools.partial(
    pl.kernel,
    out_type=(jax.ShapeDtypeStruct((N, HD), jnp.float32),
              jax.ShapeDtypeStruct((N, HD), jnp.float32)),
    mesh=_MESH,
    compiler_params=_SC_PARAMS,
    scratch_types=[
        pltpu.VMEM_SHARED((TBL, HD), jnp.float32),
        pltpu.VMEM((ZR, HD), jnp.float32),
        pltpu.VMEM((CPB, CHUNK), jnp.int32),
        pltpu.VMEM((CPB, CHUNK), jnp.int32),
    ] + [pltpu.VMEM((CHUNK, HD), jnp.float32) for _ in range(NBUF)]
      + [pltpu.SemaphoreType.DMA for _ in range(2 * NBUF)])
def _sc_agg(src2_h, dst2_h, tlo_h, thi_h, z_h, Slo_h, Shi_h, agg, zv, sidx,
            draw, *bufs):
  rows = list(bufs[:NBUF])
  gsem = list(bufs[NBUF:2 * NBUF])
  ssem = list(bufs[2 * NBUF:])
  c = lax.axis_index("c")
  s = lax.axis_index("s")

  def run(t_h, S_h):
    _zero_table(agg, zv, z_h, s)
    plsc.subcore_barrier()

    def blk(b, carry):
      roff = s * ROWS_PT + b * CPB
      pltpu.sync_copy(src2_h.at[pl.ds(roff, CPB)], sidx)
      pltpu.sync_copy(dst2_h.at[pl.ds(roff, CPB)], draw)

      gd = [None] * NBUF
      sd = [None] * NBUF

      def fire(k):
        i = k % NBUF
        gd[i] = pltpu.async_copy(t_h.at[sidx.at[k]], rows[i], gsem[i])

      for k in range(min(NBUF - 1, CPB)):
        fire(k)
      for k in range(CPB):
        nk = k + NBUF - 1
        if nk < CPB:
          if sd[nk % NBUF] is not None:
            sd[nk % NBUF].wait()
          fire(nk)
        gd[k % NBUF].wait()
        sd[k % NBUF] = pltpu.async_copy(
            rows[k % NBUF], agg.at[draw.at[k]], ssem[k % NBUF], add=True)
      for i in range(NBUF):
        k = CPB - NBUF + i
        if k >= 0 and sd[k % NBUF] is not None:
          sd[k % NBUF].wait()
      return carry

    lax.fori_loop(0, NBLK, blk, 0)
    plsc.subcore_barrier()
    _writeback(agg, S_h, s)

  @pl.when(c == 0)
  def _():
    run(tlo_h, Slo_h)

  @pl.when(c == 1)
  def _():
    run(thi_h, Shi_h)


R = 2000


def _enc_body(xc_r, we_r, be_r, wg_r, tlo_r, thi_r):
  xc = xc_r[...]
  dis = lax.rsqrt(xc[:, 12:13] + xc[:, 20:21] + 1.0)
  h = jnp.dot(xc[:, :12], we_r[...], preferred_element_type=jnp.float32)
  h = h + be_r[...]
  t = jnp.dot(h, wg_r[...], preferred_element_type=jnp.float32) * dis
  tlo_r[...] = t[:, :HD]
  thi_r[...] = t[:, HD:]


def _layer_norm(u, g, b):
  mu = jnp.mean(u, axis=-1, keepdims=True)
  var = jnp.mean((u - mu) ** 2, axis=-1, keepdims=True)
  return (u - mu) * lax.rsqrt(var + EPS) * g + b


def _mid_body(cat_r, bg_r, g_r, b_r, wg2_r, t2lo_r, t2hi_r):
  cat = cat_r[...]
  dis = lax.rsqrt(cat[:, 64:65] + cat[:, 72:73] + 1.0)
  S = cat[:, :D]
  t = cat[:, D:2 * D]
  u = (S + t) * dis + bg_r[...]
  h = jnp.maximum(_layer_norm(u, g_r[...], b_r[...]), 0.0)
  t2 = jnp.dot(h, wg2_r[...], preferred_element_type=jnp.float32) * dis
  t2lo_r[...] = t2[:, :HD]
  t2hi_r[...] = t2[:, HD:]


def _out_body(cat_r, bg_r, g_r, b_r, wref_r, bref_r, wcat_r, bcat_r, o_r):
  cat = cat_r[...]
  dis = lax.rsqrt(cat[:, 64:65] + cat[:, 72:73] + 1.0)
  S = cat[:, :D]
  t = cat[:, D:2 * D]
  u = (S + t) * dis + bg_r[...]
  h = jnp.maximum(_layer_norm(u, g_r[...], b_r[...]), 0.0)
  belief = jnp.maximum(
      jnp.dot(h, wref_r[...], preferred_element_type=jnp.float32) + bref_r[...],
      0.0)
  o_r[...] = jnp.dot(belief, wcat_r[...],
                     preferred_element_type=jnp.float32) + bcat_r[...]


def _row_spec(cols):
  return pl.BlockSpec((R, cols), lambda i: (i, 0))


def _full_spec(r, c):
  return pl.BlockSpec((r, c), lambda i: (0, 0))


def _tc_call(body, in_specs, out_cols, args):
  if isinstance(out_cols, tuple):
    out_specs = [_row_spec(cc) for cc in out_cols]
    out_shape = [jax.ShapeDtypeStruct((N, cc), jnp.float32)
                 for cc in out_cols]
  else:
    out_specs = _row_spec(out_cols)
    out_shape = jax.ShapeDtypeStruct((N, out_cols), jnp.float32)
  return pl.pallas_call(
      body,
      grid=(N // R,),
      in_specs=in_specs,
      out_specs=out_specs,
      out_shape=out_shape,
  )(*args)


def kernel(x, edge_index, W_enc, b_enc, W_g1, b_g1, ln1_g, ln1_b, W_g2, b_g2,
           ln2_g, ln2_b, W_ref, b_ref, W_q, b_q, W_f, b_f, W_v, b_v):
  src = edge_index[0].astype(jnp.int32)
  dst = edge_index[1].astype(jnp.int32)
  src2 = jnp.concatenate([src, jnp.zeros((PAD_E,), jnp.int32)]
                         ).reshape(EROWS, CHUNK)
  dst2 = jnp.concatenate([dst, jnp.full((PAD_E,), N, jnp.int32)]
                         ).reshape(EROWS, CHUNK)
  z16 = jnp.zeros((ZR, HD), jnp.float32)
  z1 = jnp.zeros((RPT, 8), jnp.float32)
  ones1 = jnp.ones((CHUNK, 8), jnp.float32)

  d0, d1 = _sc_degree(dst2, z1, ones1)

  xcat = jnp.concatenate([x, d0, d1], axis=1)
  t1lo, t1hi = _tc_call(
      _enc_body,
      [_row_spec(28), _full_spec(12, D), _full_spec(1, D), _full_spec(D, D)],
      (HD, HD),
      (xcat, W_enc, b_enc.reshape(1, D), W_g1))

  S1lo, S1hi = _sc_agg(src2, dst2, t1lo, t1hi, z16)

  cat1 = jnp.concatenate([S1lo, S1hi, t1lo, t1hi, d0, d1], axis=1)
  t2lo, t2hi = _tc_call(
      _mid_body,
      [_row_spec(80), _full_spec(1, D), _full_spec(1, D), _full_spec(1, D),
       _full_spec(D, D)],
      (HD, HD),
      (cat1, b_g1.reshape(1, D), ln1_g.reshape(1, D), ln1_b.reshape(1, D),
       W_g2))

  S2lo, S2hi = _sc_agg(src2, dst2, t2lo, t2hi, z16)

  W_cat = jnp.concatenate([W_q, W_f, W_v], axis=1)
  b_cat = jnp.concatenate([b_q, b_f, b_v]).reshape(1, -1)
  cat2 = jnp.concatenate([S2lo, S2hi, t2lo, t2hi, d0, d1], axis=1)
  out = _tc_call(
      _out_body,
      [_row_spec(80), _full_spec(1, D), _full_spec(1, D), _full_spec(1, D),
       _full_spec(D, D), _full_spec(1, D), _full_spec(D, 22),
       _full_spec(1, 22)],
      22,
      (cat2, b_g2.reshape(1, D), ln2_g.reshape(1, D), ln2_b.reshape(1, D),
       W_ref, b_ref.reshape(1, D), W_cat, b_cat))
  return out


# speedup vs baseline: 1.1749x; 1.1749x over previous
import functools

import jax
import jax.numpy as jnp
from jax import lax
from jax.experimental import pallas as pl
from jax.experimental.pallas import tpu as pltpu
from jax.experimental.pallas import tpu_sc as plsc

N = 100000
E = 1600000
D = 32
HD = D // 2
EPS = 1e-5

NC = 2
NS = 16
NW = NC * NS

TBL = 100096
RPT = TBL // NS
WB_LAST = N - (NS - 1) * RPT
ZR = 782
NZ = RPT // ZR

CHUNK = 128
CPB = 8
NBUF = 4
EROWS = 12800
PAD_E = EROWS * CHUNK - E
ROWS_PT = EROWS // NS
NBLK = ROWS_PT // CPB
ROWS_PW = EROWS // NW
DBLK = ROWS_PW // CPB

_MESH = plsc.VectorSubcoreMesh(
    core_axis_name="c", subcore_axis_name="s", num_cores=NC, num_subcores=NS)
_SC_PARAMS = pltpu.CompilerParams(use_tc_tiling_on_sc=False)


def _zero_table(agg, zv, z_h, s):
  pltpu.sync_copy(z_h, zv)
  for t in range(NZ):
    pltpu.sync_copy(zv, agg.at[pl.ds(s * RPT + t * ZR, ZR)])


def _writeback(agg, out_h, s):
  @pl.when(s < NS - 1)
  def _():
    pltpu.sync_copy(agg.at[pl.ds(s * RPT, RPT)],
                    out_h.at[pl.ds(s * RPT, RPT)])

  @pl.when(s == NS - 1)
  def _():
    pltpu.sync_copy(agg.at[pl.ds((NS - 1) * RPT, WB_LAST)],
                    out_h.at[pl.ds((NS - 1) * RPT, WB_LAST)])


@functools.partial(
    pl.kernel,
    out_type=(jax.ShapeDtypeStruct((N, 8), jnp.float32),
              jax.ShapeDtypeStruct((N, 8), jnp.float32)),
    mesh=_MESH,
    compiler_params=_SC_PARAMS,
    scratch_types=[
        pltpu.VMEM_SHARED((TBL, 8), jnp.float32),
        pltpu.VMEM((RPT, 8), jnp.float32),
        pltpu.VMEM((CHUNK, 8), jnp.float32),
        pltpu.VMEM((CPB, CHUNK), jnp.int32),
        pltpu.SemaphoreType.DMA,
    ])
def _sc_degree(dst2_h, z_h, ones_h, d0_h, d1_h, agg, zv, ov, draw, sem):
  c = lax.axis_index("c")
  s = lax.axis_index("s")
  pltpu.sync_copy(ones_h, ov)
  pltpu.sync_copy(z_h, zv)
  pltpu.sync_copy(zv, agg.at[pl.ds(s * RPT, RPT)])
  plsc.subcore_barrier()

  wid = c * NS + s

  def blk(b, carry):
    roff = wid * ROWS_PW + b * CPB
    pltpu.sync_copy(dst2_h.at[pl.ds(roff, CPB)], draw)
    descs = [pltpu.async_copy(ov, agg.at[draw.at[k]], sem, add=True)
             for k in range(CPB)]
    for d in descs:
      d.wait()
    return carry

  lax.fori_loop(0, DBLK, blk, 0)
  plsc.subcore_barrier()

  @pl.when(c == 0)
  def _():
    _writeback(agg, d0_h, s)

  @pl.when(c == 1)
  def _():
    _writeback(agg, d1_h, s)


@functools.partial(
    pl.kernel,
    out_type=(jax.ShapeDtypeStruct((N, HD), jnp.float32),
              jax.ShapeDtypeStruct((N, HD), jnp.float32)),
    mesh=_MESH,
    compiler_params=_SC_PARAMS,
    scratch_types=[
        pltpu.VMEM_SHARED((TBL, HD), jnp.float32),
        pltpu.VMEM((ZR, HD), jnp.float32),
        pltpu.VMEM((CPB, CHUNK), jnp.int32),
        pltpu.VMEM((CPB, CHUNK), jnp.int32),
    ] + [pltpu.VMEM((CHUNK, HD), jnp.float32) for _ in range(NBUF)]
      + [pltpu.SemaphoreType.DMA for _ in range(2 * NBUF)])
def _sc_agg(src2_h, dst2_h, tlo_h, thi_h, z_h, Slo_h, Shi_h, agg, zv, sidx,
            draw, *bufs):
  rows = list(bufs[:NBUF])
  gsem = list(bufs[NBUF:2 * NBUF])
  ssem = list(bufs[2 * NBUF:])
  c = lax.axis_index("c")
  s = lax.axis_index("s")

  def run(t_h, S_h):
    _zero_table(agg, zv, z_h, s)
    plsc.subcore_barrier()

    def blk(b, carry):
      roff = s * ROWS_PT + b * CPB
      pltpu.sync_copy(src2_h.at[pl.ds(roff, CPB)], sidx)
      pltpu.sync_copy(dst2_h.at[pl.ds(roff, CPB)], draw)

      gd = [None] * NBUF
      sd = [None] * NBUF

      def fire(k):
        i = k % NBUF
        gd[i] = pltpu.async_copy(t_h.at[sidx.at[k]], rows[i], gsem[i])

      for k in range(min(NBUF - 1, CPB)):
        fire(k)
      for k in range(CPB):
        nk = k + NBUF - 1
        if nk < CPB:
          if sd[nk % NBUF] is not None:
            sd[nk % NBUF].wait()
          fire(nk)
        gd[k % NBUF].wait()
        sd[k % NBUF] = pltpu.async_copy(
            rows[k % NBUF], agg.at[draw.at[k]], ssem[k % NBUF], add=True)
      for i in range(NBUF):
        k = CPB - NBUF + i
        if k >= 0 and sd[k % NBUF] is not None:
          sd[k % NBUF].wait()
      return carry

    lax.fori_loop(0, NBLK, blk, 0)
    plsc.subcore_barrier()
    _writeback(agg, S_h, s)

  @pl.when(c == 0)
  def _():
    run(tlo_h, Slo_h)

  @pl.when(c == 1)
  def _():
    run(thi_h, Shi_h)


R = 5000


def _dis(d0, d1):
  return lax.rsqrt(d0[:, 0:1] + d1[:, 0:1] + 1.0)


def _enc_body(x_r, d0_r, d1_r, we_r, be_r, wg_r, tlo_r, thi_r):
  h = jnp.dot(x_r[...], we_r[...], preferred_element_type=jnp.float32)
  h = h + be_r[...]
  t = jnp.dot(h, wg_r[...],
              preferred_element_type=jnp.float32) * _dis(d0_r[...], d1_r[...])
  tlo_r[...] = t[:, :HD]
  thi_r[...] = t[:, HD:]


def _layer_norm(u, g, b):
  mu = jnp.mean(u, axis=-1, keepdims=True)
  var = jnp.mean((u - mu) ** 2, axis=-1, keepdims=True)
  return (u - mu) * lax.rsqrt(var + EPS) * g + b


def _mid_body(Sl_r, Sh_r, tl_r, th_r, d0_r, d1_r, bg_r, g_r, b_r, wg2_r,
              t2lo_r, t2hi_r):
  dis = _dis(d0_r[...], d1_r[...])
  S = jnp.concatenate([Sl_r[...], Sh_r[...]], axis=-1)
  t = jnp.concatenate([tl_r[...], th_r[...]], axis=-1)
  u = (S + t) * dis + bg_r[...]
  h = jnp.maximum(_layer_norm(u, g_r[...], b_r[...]), 0.0)
  t2 = jnp.dot(h, wg2_r[...], preferred_element_type=jnp.float32) * dis
  t2lo_r[...] = t2[:, :HD]
  t2hi_r[...] = t2[:, HD:]


def _out_body(Sl_r, Sh_r, tl_r, th_r, d0_r, d1_r, bg_r, g_r, b_r, wref_r,
              bref_r, wcat_r, bcat_r, o_r):
  dis = _dis(d0_r[...], d1_r[...])
  S = jnp.concatenate([Sl_r[...], Sh_r[...]], axis=-1)
  t = jnp.concatenate([tl_r[...], th_r[...]], axis=-1)
  u = (S + t) * dis + bg_r[...]
  h = jnp.maximum(_layer_norm(u, g_r[...], b_r[...]), 0.0)
  belief = jnp.maximum(
      jnp.dot(h, wref_r[...], preferred_element_type=jnp.float32) + bref_r[...],
      0.0)
  o_r[...] = jnp.dot(belief, wcat_r[...],
                     preferred_element_type=jnp.float32) + bcat_r[...]


def _row_spec(cols):
  return pl.BlockSpec((R, cols), lambda i: (i, 0))


def _full_spec(r, c):
  return pl.BlockSpec((r, c), lambda i: (0, 0))


def _tc_call(body, in_specs, out_cols, args):
  if isinstance(out_cols, tuple):
    out_specs = [_row_spec(cc) for cc in out_cols]
    out_shape = [jax.ShapeDtypeStruct((N, cc), jnp.float32)
                 for cc in out_cols]
  else:
    out_specs = _row_spec(out_cols)
    out_shape = jax.ShapeDtypeStruct((N, out_cols), jnp.float32)
  return pl.pallas_call(
      body,
      grid=(N // R,),
      in_specs=in_specs,
      out_specs=out_specs,
      out_shape=out_shape,
  )(*args)


def kernel(x, edge_index, W_enc, b_enc, W_g1, b_g1, ln1_g, ln1_b, W_g2, b_g2,
           ln2_g, ln2_b, W_ref, b_ref, W_q, b_q, W_f, b_f, W_v, b_v):
  src = edge_index[0].astype(jnp.int32)
  dst = edge_index[1].astype(jnp.int32)
  src2 = jnp.concatenate([src, jnp.zeros((PAD_E,), jnp.int32)]
                         ).reshape(EROWS, CHUNK)
  dst2 = jnp.concatenate([dst, jnp.full((PAD_E,), N, jnp.int32)]
                         ).reshape(EROWS, CHUNK)
  z16 = jnp.zeros((ZR, HD), jnp.float32)
  z1 = jnp.zeros((RPT, 8), jnp.float32)
  ones1 = jnp.ones((CHUNK, 8), jnp.float32)

  d0, d1 = _sc_degree(dst2, z1, ones1)

  t1lo, t1hi = _tc_call(
      _enc_body,
      [_row_spec(12), _row_spec(8), _row_spec(8), _full_spec(12, D),
       _full_spec(1, D), _full_spec(D, D)],
      (HD, HD),
      (x, d0, d1, W_enc, b_enc.reshape(1, D), W_g1))

  S1lo, S1hi = _sc_agg(src2, dst2, t1lo, t1hi, z16)

  t2lo, t2hi = _tc_call(
      _mid_body,
      [_row_spec(HD), _row_spec(HD), _row_spec(HD), _row_spec(HD),
       _row_spec(8), _row_spec(8), _full_spec(1, D), _full_spec(1, D),
       _full_spec(1, D), _full_spec(D, D)],
      (HD, HD),
      (S1lo, S1hi, t1lo, t1hi, d0, d1, b_g1.reshape(1, D),
       ln1_g.reshape(1, D), ln1_b.reshape(1, D), W_g2))

  S2lo, S2hi = _sc_agg(src2, dst2, t2lo, t2hi, z16)

  W_cat = jnp.concatenate([W_q, W_f, W_v], axis=1)
  b_cat = jnp.concatenate([b_q, b_f, b_v]).reshape(1, -1)
  out = _tc_call(
      _out_body,
      [_row_spec(HD), _row_spec(HD), _row_spec(HD), _row_spec(HD),
       _row_spec(8), _row_spec(8), _full_spec(1, D), _full_spec(1, D),
       _full_spec(1, D), _full_spec(D, D), _full_spec(1, D),
       _full_spec(D, 22), _full_spec(1, 22)],
      22,
      (S2lo, S2hi, t2lo, t2hi, d0, d1, b_g2.reshape(1, D),
       ln2_g.reshape(1, D), ln2_b.reshape(1, D), W_ref, b_ref.reshape(1, D),
       W_cat, b_cat))
  return out


# speedup vs baseline: 1.5061x; 1.2819x over previous
import functools

import jax
import jax.numpy as jnp
from jax import lax
from jax.experimental import pallas as pl
from jax.experimental.pallas import tpu as pltpu
from jax.experimental.pallas import tpu_sc as plsc

N = 100000
E = 1600000
D = 32
HD = D // 2
EPS = 1e-5

NC = 2
NS = 16
NW = NC * NS

TBL = 100096
RPT = TBL // NS
WB_LAST = N - (NS - 1) * RPT
ZR = 782
NZ = RPT // ZR

CHUNK = 128
CPB = 8
BLKE = CPB * CHUNK
NBUF = 4
EPT = E // NS
NBLK = EPT // BLKE
TAIL = EPT - NBLK * BLKE
EPW = E // NW
DBLK = EPW // BLKE
DTAIL = EPW - DBLK * BLKE

_FULL = [(k * CHUNK, CHUNK) for k in range(CPB)]


def _chunks_of(total):
  out = []
  off = 0
  while off < total:
    sz = min(CHUNK, total - off)
    out.append((off, sz))
    off += sz
  return out

_MESH = plsc.VectorSubcoreMesh(
    core_axis_name="c", subcore_axis_name="s", num_cores=NC, num_subcores=NS)
_SC_PARAMS = pltpu.CompilerParams(use_tc_tiling_on_sc=False)


def _zero_table(agg, zv, z_h, s):
  pltpu.sync_copy(z_h, zv)
  for t in range(NZ):
    pltpu.sync_copy(zv, agg.at[pl.ds(s * RPT + t * ZR, ZR)])


def _writeback(agg, out_h, s):
  @pl.when(s < NS - 1)
  def _():
    pltpu.sync_copy(agg.at[pl.ds(s * RPT, RPT)],
                    out_h.at[pl.ds(s * RPT, RPT)])

  @pl.when(s == NS - 1)
  def _():
    pltpu.sync_copy(agg.at[pl.ds((NS - 1) * RPT, WB_LAST)],
                    out_h.at[pl.ds((NS - 1) * RPT, WB_LAST)])


@functools.partial(
    pl.kernel,
    out_type=(jax.ShapeDtypeStruct((N, 8), jnp.float32),
              jax.ShapeDtypeStruct((N, 8), jnp.float32)),
    mesh=_MESH,
    compiler_params=_SC_PARAMS,
    scratch_types=[
        pltpu.VMEM_SHARED((TBL, 8), jnp.float32),
        pltpu.VMEM((RPT, 8), jnp.float32),
        pltpu.VMEM((CHUNK, 8), jnp.float32),
        pltpu.VMEM((BLKE,), jnp.int32),
        pltpu.SemaphoreType.DMA,
    ])
def _sc_degree(dst_h, z_h, ones_h, d0_h, d1_h, agg, zv, ov, draw, sem):
  c = lax.axis_index("c")
  s = lax.axis_index("s")
  pltpu.sync_copy(ones_h, ov)
  pltpu.sync_copy(z_h, zv)
  pltpu.sync_copy(zv, agg.at[pl.ds(s * RPT, RPT)])
  plsc.subcore_barrier()

  ebase = (c * NS + s) * EPW

  def do_block(eoff, chunks, nidx):
    pltpu.sync_copy(dst_h.at[pl.ds(eoff, nidx)], draw.at[pl.ds(0, nidx)])
    descs = [pltpu.async_copy(ov.at[pl.ds(0, sz)],
                              agg.at[draw.at[pl.ds(off, sz)]], sem, add=True)
             for off, sz in chunks]
    for d in descs:
      d.wait()

  def blk(b, carry):
    do_block(ebase + b * BLKE, _FULL, BLKE)
    return carry

  lax.fori_loop(0, DBLK, blk, 0)
  do_block(ebase + DBLK * BLKE, _chunks_of(DTAIL), DTAIL)
  plsc.subcore_barrier()

  @pl.when(c == 0)
  def _():
    _writeback(agg, d0_h, s)

  @pl.when(c == 1)
  def _():
    _writeback(agg, d1_h, s)


@functools.partial(
    pl.kernel,
    out_type=(jax.ShapeDtypeStruct((N, HD), jnp.float32),
              jax.ShapeDtypeStruct((N, HD), jnp.float32)),
    mesh=_MESH,
    compiler_params=_SC_PARAMS,
    scratch_types=[
        pltpu.VMEM_SHARED((TBL, HD), jnp.float32),
        pltpu.VMEM((ZR, HD), jnp.float32),
        pltpu.VMEM((BLKE,), jnp.int32),
        pltpu.VMEM((BLKE,), jnp.int32),
    ] + [pltpu.VMEM((CHUNK, HD), jnp.float32) for _ in range(NBUF)]
      + [pltpu.SemaphoreType.DMA for _ in range(2 * NBUF)])
def _sc_agg(src_h, dst_h, tlo_h, thi_h, z_h, Slo_h, Shi_h, agg, zv, sidx,
            draw, *bufs):
  rows = list(bufs[:NBUF])
  gsem = list(bufs[NBUF:2 * NBUF])
  ssem = list(bufs[2 * NBUF:])
  c = lax.axis_index("c")
  s = lax.axis_index("s")

  def run(t_h, S_h):
    _zero_table(agg, zv, z_h, s)
    plsc.subcore_barrier()
    ebase = s * EPT

    def do_block(eoff, chunks, nidx):
      pltpu.sync_copy(src_h.at[pl.ds(eoff, nidx)], sidx.at[pl.ds(0, nidx)])
      pltpu.sync_copy(dst_h.at[pl.ds(eoff, nidx)], draw.at[pl.ds(0, nidx)])
      n = len(chunks)
      gd = [None] * NBUF
      sd = [None] * NBUF

      def fire(k):
        off, sz = chunks[k]
        i = k % NBUF
        gd[i] = pltpu.async_copy(
            t_h.at[sidx.at[pl.ds(off, sz)]], rows[i].at[pl.ds(0, sz)],
            gsem[i])

      for k in range(min(NBUF - 1, n)):
        fire(k)
      for k in range(n):
        off, sz = chunks[k]
        nk = k + NBUF - 1
        if nk < n:
          if sd[nk % NBUF] is not None:
            sd[nk % NBUF].wait()
          fire(nk)
        gd[k % NBUF].wait()
        sd[k % NBUF] = pltpu.async_copy(
            rows[k % NBUF].at[pl.ds(0, sz)],
            agg.at[draw.at[pl.ds(off, sz)]], ssem[k % NBUF], add=True)
      for i in range(NBUF):
        k = n - NBUF + i
        if k >= 0 and sd[k % NBUF] is not None:
          sd[k % NBUF].wait()

    def blk(b, carry):
      do_block(ebase + b * BLKE, _FULL, BLKE)
      return carry

    lax.fori_loop(0, NBLK, blk, 0)
    do_block(ebase + NBLK * BLKE, _chunks_of(TAIL), TAIL)
    plsc.subcore_barrier()
    _writeback(agg, S_h, s)

  @pl.when(c == 0)
  def _():
    run(tlo_h, Slo_h)

  @pl.when(c == 1)
  def _():
    run(thi_h, Shi_h)


R = 5000


def _dis(d0, d1):
  return lax.rsqrt(d0[:, 0:1] + d1[:, 0:1] + 1.0)


def _enc_body(x_r, d0_r, d1_r, we_r, be_r, wg_r, tlo_r, thi_r):
  h = jnp.dot(x_r[...], we_r[...], preferred_element_type=jnp.float32)
  h = h + be_r[...]
  t = jnp.dot(h, wg_r[...],
              preferred_element_type=jnp.float32) * _dis(d0_r[...], d1_r[...])
  tlo_r[...] = t[:, :HD]
  thi_r[...] = t[:, HD:]


def _layer_norm(u, g, b):
  mu = jnp.mean(u, axis=-1, keepdims=True)
  var = jnp.mean((u - mu) ** 2, axis=-1, keepdims=True)
  return (u - mu) * lax.rsqrt(var + EPS) * g + b


def _mid_body(Sl_r, Sh_r, tl_r, th_r, d0_r, d1_r, bg_r, g_r, b_r, wg2_r,
              t2lo_r, t2hi_r):
  dis = _dis(d0_r[...], d1_r[...])
  S = jnp.concatenate([Sl_r[...], Sh_r[...]], axis=-1)
  t = jnp.concatenate([tl_r[...], th_r[...]], axis=-1)
  u = (S + t) * dis + bg_r[...]
  h = jnp.maximum(_layer_norm(u, g_r[...], b_r[...]), 0.0)
  t2 = jnp.dot(h, wg2_r[...], preferred_element_type=jnp.float32) * dis
  t2lo_r[...] = t2[:, :HD]
  t2hi_r[...] = t2[:, HD:]


def _out_body(Sl_r, Sh_r, tl_r, th_r, d0_r, d1_r, bg_r, g_r, b_r, wref_r,
              bref_r, wcat_r, bcat_r, o_r):
  dis = _dis(d0_r[...], d1_r[...])
  S = jnp.concatenate([Sl_r[...], Sh_r[...]], axis=-1)
  t = jnp.concatenate([tl_r[...], th_r[...]], axis=-1)
  u = (S + t) * dis + bg_r[...]
  h = jnp.maximum(_layer_norm(u, g_r[...], b_r[...]), 0.0)
  belief = jnp.maximum(
      jnp.dot(h, wref_r[...], preferred_element_type=jnp.float32) + bref_r[...],
      0.0)
  o_r[...] = jnp.dot(belief, wcat_r[...],
                     preferred_element_type=jnp.float32) + bcat_r[...]


def _row_spec(cols):
  return pl.BlockSpec((R, cols), lambda i: (i, 0))


def _full_spec(r, c):
  return pl.BlockSpec((r, c), lambda i: (0, 0))


def _tc_call(body, in_specs, out_cols, args):
  if isinstance(out_cols, tuple):
    out_specs = [_row_spec(cc) for cc in out_cols]
    out_shape = [jax.ShapeDtypeStruct((N, cc), jnp.float32)
                 for cc in out_cols]
  else:
    out_specs = _row_spec(out_cols)
    out_shape = jax.ShapeDtypeStruct((N, out_cols), jnp.float32)
  return pl.pallas_call(
      body,
      grid=(N // R,),
      in_specs=in_specs,
      out_specs=out_specs,
      out_shape=out_shape,
  )(*args)


def kernel(x, edge_index, W_enc, b_enc, W_g1, b_g1, ln1_g, ln1_b, W_g2, b_g2,
           ln2_g, ln2_b, W_ref, b_ref, W_q, b_q, W_f, b_f, W_v, b_v):
  src = edge_index[0].astype(jnp.int32)
  dst = edge_index[1].astype(jnp.int32)
  z16 = jnp.zeros((ZR, HD), jnp.float32)
  z1 = jnp.zeros((RPT, 8), jnp.float32)
  ones1 = jnp.ones((CHUNK, 8), jnp.float32)

  d0, d1 = _sc_degree(dst, z1, ones1)

  t1lo, t1hi = _tc_call(
      _enc_body,
      [_row_spec(12), _row_spec(8), _row_spec(8), _full_spec(12, D),
       _full_spec(1, D), _full_spec(D, D)],
      (HD, HD),
      (x, d0, d1, W_enc, b_enc.reshape(1, D), W_g1))

  S1lo, S1hi = _sc_agg(src, dst, t1lo, t1hi, z16)

  t2lo, t2hi = _tc_call(
      _mid_body,
      [_row_spec(HD), _row_spec(HD), _row_spec(HD), _row_spec(HD),
       _row_spec(8), _row_spec(8), _full_spec(1, D), _full_spec(1, D),
       _full_spec(1, D), _full_spec(D, D)],
      (HD, HD),
      (S1lo, S1hi, t1lo, t1hi, d0, d1, b_g1.reshape(1, D),
       ln1_g.reshape(1, D), ln1_b.reshape(1, D), W_g2))

  S2lo, S2hi = _sc_agg(src, dst, t2lo, t2hi, z16)

  W_cat = jnp.concatenate([W_q, W_f, W_v], axis=1)
  b_cat = jnp.concatenate([b_q, b_f, b_v]).reshape(1, -1)
  out = _tc_call(
      _out_body,
      [_row_spec(HD), _row_spec(HD), _row_spec(HD), _row_spec(HD),
       _row_spec(8), _row_spec(8), _full_spec(1, D), _full_spec(1, D),
       _full_spec(1, D), _full_spec(D, D), _full_spec(1, D),
       _full_spec(D, 22), _full_spec(1, 22)],
      22,
      (S2lo, S2hi, t2lo, t2hi, d0, d1, b_g2.reshape(1, D),
       ln2_g.reshape(1, D), ln2_b.reshape(1, D), W_ref, b_ref.reshape(1, D),
       W_cat, b_cat))
  return out


# speedup vs baseline: 1.6759x; 1.1128x over previous
import functools

import jax
import jax.numpy as jnp
from jax import lax
from jax.experimental import pallas as pl
from jax.experimental.pallas import tpu as pltpu
from jax.experimental.pallas import tpu_sc as plsc

N = 100000
E = 1600000
D = 32
HD = D // 2
EPS = 1e-5

NC = 2
NS = 16
NW = NC * NS

TBL = 100096
RPT = TBL // NS
WB_LAST = N - (NS - 1) * RPT
ZR = 782
NZ = RPT // ZR

CHUNK = 128
CPB = 16
BLKE = CPB * CHUNK
NBUF = 4
EPT = E // NS
NBLK = EPT // BLKE
TAIL = EPT - NBLK * BLKE
EPW = E // NW
DBLK = EPW // BLKE
DTAIL = EPW - DBLK * BLKE

_FULL = [(k * CHUNK, CHUNK) for k in range(CPB)]


def _chunks_of(total):
  out = []
  off = 0
  while off < total:
    sz = min(CHUNK, total - off)
    out.append((off, sz))
    off += sz
  return out

_MESH = plsc.VectorSubcoreMesh(
    core_axis_name="c", subcore_axis_name="s", num_cores=NC, num_subcores=NS)
_SC_PARAMS = pltpu.CompilerParams(use_tc_tiling_on_sc=False)


def _zero_table(agg, zv, z_h, s):
  pltpu.sync_copy(z_h, zv)
  for t in range(NZ):
    pltpu.sync_copy(zv, agg.at[pl.ds(s * RPT + t * ZR, ZR)])


def _writeback(agg, out_h, s):
  @pl.when(s < NS - 1)
  def _():
    pltpu.sync_copy(agg.at[pl.ds(s * RPT, RPT)],
                    out_h.at[pl.ds(s * RPT, RPT)])

  @pl.when(s == NS - 1)
  def _():
    pltpu.sync_copy(agg.at[pl.ds((NS - 1) * RPT, WB_LAST)],
                    out_h.at[pl.ds((NS - 1) * RPT, WB_LAST)])


@functools.partial(
    pl.kernel,
    out_type=(jax.ShapeDtypeStruct((N, 8), jnp.float32),
              jax.ShapeDtypeStruct((N, 8), jnp.float32)),
    mesh=_MESH,
    compiler_params=_SC_PARAMS,
    scratch_types=[
        pltpu.VMEM_SHARED((TBL, 8), jnp.float32),
        pltpu.VMEM((RPT, 8), jnp.float32),
        pltpu.VMEM((CHUNK, 8), jnp.float32),
        pltpu.VMEM((BLKE,), jnp.int32),
        pltpu.SemaphoreType.DMA,
    ])
def _sc_degree(dst_h, z_h, ones_h, d0_h, d1_h, agg, zv, ov, draw, sem):
  c = lax.axis_index("c")
  s = lax.axis_index("s")
  pltpu.sync_copy(ones_h, ov)
  pltpu.sync_copy(z_h, zv)
  pltpu.sync_copy(zv, agg.at[pl.ds(s * RPT, RPT)])
  plsc.subcore_barrier()

  ebase = (c * NS + s) * EPW

  def do_block(eoff, chunks, nidx):
    pltpu.sync_copy(dst_h.at[pl.ds(eoff, nidx)], draw.at[pl.ds(0, nidx)])
    descs = [pltpu.async_copy(ov.at[pl.ds(0, sz)],
                              agg.at[draw.at[pl.ds(off, sz)]], sem, add=True)
             for off, sz in chunks]
    for d in descs:
      d.wait()

  def blk(b, carry):
    do_block(ebase + b * BLKE, _FULL, BLKE)
    return carry

  lax.fori_loop(0, DBLK, blk, 0)
  do_block(ebase + DBLK * BLKE, _chunks_of(DTAIL), DTAIL)
  plsc.subcore_barrier()

  @pl.when(c == 0)
  def _():
    _writeback(agg, d0_h, s)

  @pl.when(c == 1)
  def _():
    _writeback(agg, d1_h, s)


@functools.partial(
    pl.kernel,
    out_type=(jax.ShapeDtypeStruct((N, HD), jnp.float32),
              jax.ShapeDtypeStruct((N, HD), jnp.float32)),
    mesh=_MESH,
    compiler_params=_SC_PARAMS,
    scratch_types=[
        pltpu.VMEM_SHARED((TBL, HD), jnp.float32),
        pltpu.VMEM((ZR, HD), jnp.float32),
        pltpu.VMEM((BLKE,), jnp.int32),
        pltpu.VMEM((BLKE,), jnp.int32),
    ] + [pltpu.VMEM((CHUNK, HD), jnp.float32) for _ in range(NBUF)]
      + [pltpu.SemaphoreType.DMA for _ in range(2 * NBUF)])
def _sc_agg(src_h, dst_h, tlo_h, thi_h, z_h, Slo_h, Shi_h, agg, zv, sidx,
            draw, *bufs):
  rows = list(bufs[:NBUF])
  gsem = list(bufs[NBUF:2 * NBUF])
  ssem = list(bufs[2 * NBUF:])
  c = lax.axis_index("c")
  s = lax.axis_index("s")

  def run(t_h, S_h):
    _zero_table(agg, zv, z_h, s)
    plsc.subcore_barrier()
    ebase = s * EPT

    def do_block(eoff, chunks, nidx):
      pltpu.sync_copy(src_h.at[pl.ds(eoff, nidx)], sidx.at[pl.ds(0, nidx)])
      pltpu.sync_copy(dst_h.at[pl.ds(eoff, nidx)], draw.at[pl.ds(0, nidx)])
      n = len(chunks)
      gd = [None] * NBUF
      sd = [None] * NBUF

      def fire(k):
        off, sz = chunks[k]
        i = k % NBUF
        gd[i] = pltpu.async_copy(
            t_h.at[sidx.at[pl.ds(off, sz)]], rows[i].at[pl.ds(0, sz)],
            gsem[i])

      for k in range(min(NBUF - 1, n)):
        fire(k)
      for k in range(n):
        off, sz = chunks[k]
        nk = k + NBUF - 1
        if nk < n:
          if sd[nk % NBUF] is not None:
            sd[nk % NBUF].wait()
          fire(nk)
        gd[k % NBUF].wait()
        sd[k % NBUF] = pltpu.async_copy(
            rows[k % NBUF].at[pl.ds(0, sz)],
            agg.at[draw.at[pl.ds(off, sz)]], ssem[k % NBUF], add=True)
      for i in range(NBUF):
        k = n - NBUF + i
        if k >= 0 and sd[k % NBUF] is not None:
          sd[k % NBUF].wait()

    def blk(b, carry):
      do_block(ebase + b * BLKE, _FULL, BLKE)
      return carry

    lax.fori_loop(0, NBLK, blk, 0)
    do_block(ebase + NBLK * BLKE, _chunks_of(TAIL), TAIL)
    plsc.subcore_barrier()
    _writeback(agg, S_h, s)

  @pl.when(c == 0)
  def _():
    run(tlo_h, Slo_h)

  @pl.when(c == 1)
  def _():
    run(thi_h, Shi_h)


R = 5000


def _dis(d0, d1):
  return lax.rsqrt(d0[:, 0:1] + d1[:, 0:1] + 1.0)


def _enc_body(x_r, d0_r, d1_r, we_r, be_r, wg_r, tlo_r, thi_r, dis_r):
  h = jnp.dot(x_r[...], we_r[...], preferred_element_type=jnp.float32)
  h = h + be_r[...]
  dis = _dis(d0_r[...], d1_r[...])
  t = jnp.dot(h, wg_r[...], preferred_element_type=jnp.float32) * dis
  tlo_r[...] = t[:, :HD]
  thi_r[...] = t[:, HD:]
  dis_r[...] = jnp.broadcast_to(dis, dis_r.shape)


def _layer_norm(u, g, b):
  mu = jnp.mean(u, axis=-1, keepdims=True)
  var = jnp.mean((u - mu) ** 2, axis=-1, keepdims=True)
  return (u - mu) * lax.rsqrt(var + EPS) * g + b


def _mid_body(Sl_r, Sh_r, tl_r, th_r, dis_r, bg_r, g_r, b_r, wg2_r,
              t2lo_r, t2hi_r):
  dis = dis_r[:, 0:1]
  S = jnp.concatenate([Sl_r[...], Sh_r[...]], axis=-1)
  t = jnp.concatenate([tl_r[...], th_r[...]], axis=-1)
  u = (S + t) * dis + bg_r[...]
  h = jnp.maximum(_layer_norm(u, g_r[...], b_r[...]), 0.0)
  t2 = jnp.dot(h, wg2_r[...], preferred_element_type=jnp.float32) * dis
  t2lo_r[...] = t2[:, :HD]
  t2hi_r[...] = t2[:, HD:]


def _out_body(Sl_r, Sh_r, tl_r, th_r, dis_r, bg_r, g_r, b_r, wref_r,
              bref_r, wcat_r, bcat_r, o_r):
  dis = dis_r[:, 0:1]
  S = jnp.concatenate([Sl_r[...], Sh_r[...]], axis=-1)
  t = jnp.concatenate([tl_r[...], th_r[...]], axis=-1)
  u = (S + t) * dis + bg_r[...]
  h = jnp.maximum(_layer_norm(u, g_r[...], b_r[...]), 0.0)
  belief = jnp.maximum(
      jnp.dot(h, wref_r[...], preferred_element_type=jnp.float32) + bref_r[...],
      0.0)
  o_r[...] = jnp.dot(belief, wcat_r[...],
                     preferred_element_type=jnp.float32) + bcat_r[...]


def _row_spec(cols):
  return pl.BlockSpec((R, cols), lambda i: (i, 0))


def _full_spec(r, c):
  return pl.BlockSpec((r, c), lambda i: (0, 0))


def _tc_call(body, in_specs, out_cols, args):
  if isinstance(out_cols, tuple):
    out_specs = [_row_spec(cc) for cc in out_cols]
    out_shape = [jax.ShapeDtypeStruct((N, cc), jnp.float32)
                 for cc in out_cols]
  else:
    out_specs = _row_spec(out_cols)
    out_shape = jax.ShapeDtypeStruct((N, out_cols), jnp.float32)
  return pl.pallas_call(
      body,
      grid=(N // R,),
      in_specs=in_specs,
      out_specs=out_specs,
      out_shape=out_shape,
  )(*args)


def kernel(x, edge_index, W_enc, b_enc, W_g1, b_g1, ln1_g, ln1_b, W_g2, b_g2,
           ln2_g, ln2_b, W_ref, b_ref, W_q, b_q, W_f, b_f, W_v, b_v):
  src = edge_index[0].astype(jnp.int32)
  dst = edge_index[1].astype(jnp.int32)
  z16 = jnp.zeros((ZR, HD), jnp.float32)
  z1 = jnp.zeros((RPT, 8), jnp.float32)
  ones1 = jnp.ones((CHUNK, 8), jnp.float32)

  d0, d1 = _sc_degree(dst, z1, ones1)

  t1lo, t1hi, disv = _tc_call(
      _enc_body,
      [_row_spec(12), _row_spec(8), _row_spec(8), _full_spec(12, D),
       _full_spec(1, D), _full_spec(D, D)],
      (HD, HD, 8),
      (x, d0, d1, W_enc, b_enc.reshape(1, D), W_g1))

  S1lo, S1hi = _sc_agg(src, dst, t1lo, t1hi, z16)

  t2lo, t2hi = _tc_call(
      _mid_body,
      [_row_spec(HD), _row_spec(HD), _row_spec(HD), _row_spec(HD),
       _row_spec(8), _full_spec(1, D), _full_spec(1, D),
       _full_spec(1, D), _full_spec(D, D)],
      (HD, HD),
      (S1lo, S1hi, t1lo, t1hi, disv, b_g1.reshape(1, D),
       ln1_g.reshape(1, D), ln1_b.reshape(1, D), W_g2))

  S2lo, S2hi = _sc_agg(src, dst, t2lo, t2hi, z16)

  W_cat = jnp.concatenate([W_q, W_f, W_v], axis=1)
  b_cat = jnp.concatenate([b_q, b_f, b_v]).reshape(1, -1)
  out = _tc_call(
      _out_body,
      [_row_spec(HD), _row_spec(HD), _row_spec(HD), _row_spec(HD),
       _row_spec(8), _full_spec(1, D), _full_spec(1, D),
       _full_spec(1, D), _full_spec(D, D), _full_spec(1, D),
       _full_spec(D, 22), _full_spec(1, 22)],
      22,
      (S2lo, S2hi, t2lo, t2hi, disv, b_g2.reshape(1, D),
       ln2_g.reshape(1, D), ln2_b.reshape(1, D), W_ref, b_ref.reshape(1, D),
       W_cat, b_cat))
  return out


# speedup vs baseline: 1.7835x; 1.0642x over previous
import functools

import jax
import jax.numpy as jnp
from jax import lax
from jax.experimental import pallas as pl
from jax.experimental.pallas import tpu as pltpu
from jax.experimental.pallas import tpu_sc as plsc

N = 100000
E = 1600000
D = 32
HD = D // 2
EPS = 1e-5

NC = 2
NS = 16
NW = NC * NS

TBL = 100096
RPT = TBL // NS
WB_LAST = N - (NS - 1) * RPT
ZR = 782
NZ = RPT // ZR

CHUNK = 128
CPB = 16
BLKE = CPB * CHUNK
NBUF = 6
EPT = E // NS
NBLK = EPT // BLKE
TAIL = EPT - NBLK * BLKE
EPW = E // NW
DBLK = EPW // BLKE
DTAIL = EPW - DBLK * BLKE

_FULL = [(k * CHUNK, CHUNK) for k in range(CPB)]


def _chunks_of(total):
  out = []
  off = 0
  while off < total:
    sz = min(CHUNK, total - off)
    out.append((off, sz))
    off += sz
  return out

_MESH = plsc.VectorSubcoreMesh(
    core_axis_name="c", subcore_axis_name="s", num_cores=NC, num_subcores=NS)
_SC_PARAMS = pltpu.CompilerParams(use_tc_tiling_on_sc=False)


def _zero_table(agg, zv, z_h, s):
  pltpu.sync_copy(z_h, zv)
  for t in range(NZ):
    pltpu.sync_copy(zv, agg.at[pl.ds(s * RPT + t * ZR, ZR)])


def _writeback(agg, out_h, s):
  @pl.when(s < NS - 1)
  def _():
    pltpu.sync_copy(agg.at[pl.ds(s * RPT, RPT)],
                    out_h.at[pl.ds(s * RPT, RPT)])

  @pl.when(s == NS - 1)
  def _():
    pltpu.sync_copy(agg.at[pl.ds((NS - 1) * RPT, WB_LAST)],
                    out_h.at[pl.ds((NS - 1) * RPT, WB_LAST)])


@functools.partial(
    pl.kernel,
    out_type=(jax.ShapeDtypeStruct((N, 8), jnp.float32),
              jax.ShapeDtypeStruct((N, 8), jnp.float32)),
    mesh=_MESH,
    compiler_params=_SC_PARAMS,
    scratch_types=[
        pltpu.VMEM_SHARED((TBL, 8), jnp.float32),
        pltpu.VMEM((RPT, 8), jnp.float32),
        pltpu.VMEM((CHUNK, 8), jnp.float32),
        pltpu.VMEM((BLKE,), jnp.int32),
        pltpu.SemaphoreType.DMA,
    ])
def _sc_degree(dst_h, z_h, ones_h, d0_h, d1_h, agg, zv, ov, draw, sem):
  c = lax.axis_index("c")
  s = lax.axis_index("s")
  pltpu.sync_copy(ones_h, ov)
  pltpu.sync_copy(z_h, zv)
  pltpu.sync_copy(zv, agg.at[pl.ds(s * RPT, RPT)])
  plsc.subcore_barrier()

  ebase = (c * NS + s) * EPW

  def do_block(eoff, chunks, nidx):
    pltpu.sync_copy(dst_h.at[pl.ds(eoff, nidx)], draw.at[pl.ds(0, nidx)])
    descs = [pltpu.async_copy(ov.at[pl.ds(0, sz)],
                              agg.at[draw.at[pl.ds(off, sz)]], sem, add=True)
             for off, sz in chunks]
    for d in descs:
      d.wait()

  def blk(b, carry):
    do_block(ebase + b * BLKE, _FULL, BLKE)
    return carry

  lax.fori_loop(0, DBLK, blk, 0)
  do_block(ebase + DBLK * BLKE, _chunks_of(DTAIL), DTAIL)
  plsc.subcore_barrier()

  @pl.when(c == 0)
  def _():
    _writeback(agg, d0_h, s)

  @pl.when(c == 1)
  def _():
    _writeback(agg, d1_h, s)


@functools.partial(
    pl.kernel,
    out_type=(jax.ShapeDtypeStruct((N, HD), jnp.float32),
              jax.ShapeDtypeStruct((N, HD), jnp.float32)),
    mesh=_MESH,
    compiler_params=_SC_PARAMS,
    scratch_types=[
        pltpu.VMEM_SHARED((TBL, HD), jnp.float32),
        pltpu.VMEM((ZR, HD), jnp.float32),
        pltpu.VMEM((BLKE,), jnp.int32),
        pltpu.VMEM((BLKE,), jnp.int32),
    ] + [pltpu.VMEM((CHUNK, HD), jnp.float32) for _ in range(NBUF)]
      + [pltpu.SemaphoreType.DMA for _ in range(2 * NBUF)])
def _sc_agg(src_h, dst_h, tlo_h, thi_h, z_h, Slo_h, Shi_h, agg, zv, sidx,
            draw, *bufs):
  rows = list(bufs[:NBUF])
  gsem = list(bufs[NBUF:2 * NBUF])
  ssem = list(bufs[2 * NBUF:])
  c = lax.axis_index("c")
  s = lax.axis_index("s")

  def run(t_h, S_h):
    _zero_table(agg, zv, z_h, s)
    plsc.subcore_barrier()
    ebase = s * EPT

    def do_block(eoff, chunks, nidx):
      pltpu.sync_copy(src_h.at[pl.ds(eoff, nidx)], sidx.at[pl.ds(0, nidx)])
      pltpu.sync_copy(dst_h.at[pl.ds(eoff, nidx)], draw.at[pl.ds(0, nidx)])
      n = len(chunks)
      gd = [None] * NBUF
      sd = [None] * NBUF

      def fire(k):
        off, sz = chunks[k]
        i = k % NBUF
        gd[i] = pltpu.async_copy(
            t_h.at[sidx.at[pl.ds(off, sz)]], rows[i].at[pl.ds(0, sz)],
            gsem[i])

      for k in range(min(NBUF - 1, n)):
        fire(k)
      for k in range(n):
        off, sz = chunks[k]
        nk = k + NBUF - 1
        if nk < n:
          if sd[nk % NBUF] is not None:
            sd[nk % NBUF].wait()
          fire(nk)
        gd[k % NBUF].wait()
        sd[k % NBUF] = pltpu.async_copy(
            rows[k % NBUF].at[pl.ds(0, sz)],
            agg.at[draw.at[pl.ds(off, sz)]], ssem[k % NBUF], add=True)
      for i in range(NBUF):
        k = n - NBUF + i
        if k >= 0 and sd[k % NBUF] is not None:
          sd[k % NBUF].wait()

    def blk(b, carry):
      do_block(ebase + b * BLKE, _FULL, BLKE)
      return carry

    lax.fori_loop(0, NBLK, blk, 0)
    do_block(ebase + NBLK * BLKE, _chunks_of(TAIL), TAIL)
    plsc.subcore_barrier()
    _writeback(agg, S_h, s)

  @pl.when(c == 0)
  def _():
    run(tlo_h, Slo_h)

  @pl.when(c == 1)
  def _():
    run(thi_h, Shi_h)


R = 5000


def _dis(d0, d1):
  return lax.rsqrt(d0[:, 0:1] + d1[:, 0:1] + 1.0)


def _enc_body(x_r, d0_r, d1_r, we_r, be_r, wg_r, tlo_r, thi_r, dis_r):
  h = jnp.dot(x_r[...], we_r[...], preferred_element_type=jnp.float32)
  h = h + be_r[...]
  dis = _dis(d0_r[...], d1_r[...])
  t = jnp.dot(h, wg_r[...], preferred_element_type=jnp.float32) * dis
  tlo_r[...] = t[:, :HD]
  thi_r[...] = t[:, HD:]
  dis_r[...] = jnp.broadcast_to(dis, dis_r.shape)


def _layer_norm(u, g, b):
  mu = jnp.mean(u, axis=-1, keepdims=True)
  var = jnp.mean((u - mu) ** 2, axis=-1, keepdims=True)
  return (u - mu) * lax.rsqrt(var + EPS) * g + b


def _mid_body(Sl_r, Sh_r, tl_r, th_r, dis_r, bg_r, g_r, b_r, wg2_r,
              t2lo_r, t2hi_r):
  dis = dis_r[:, 0:1]
  S = jnp.concatenate([Sl_r[...], Sh_r[...]], axis=-1)
  t = jnp.concatenate([tl_r[...], th_r[...]], axis=-1)
  u = (S + t) * dis + bg_r[...]
  h = jnp.maximum(_layer_norm(u, g_r[...], b_r[...]), 0.0)
  t2 = jnp.dot(h, wg2_r[...], preferred_element_type=jnp.float32) * dis
  t2lo_r[...] = t2[:, :HD]
  t2hi_r[...] = t2[:, HD:]


def _out_body(Sl_r, Sh_r, tl_r, th_r, dis_r, bg_r, g_r, b_r, wref_r,
              bref_r, wcat_r, bcat_r, o_r):
  dis = dis_r[:, 0:1]
  S = jnp.concatenate([Sl_r[...], Sh_r[...]], axis=-1)
  t = jnp.concatenate([tl_r[...], th_r[...]], axis=-1)
  u = (S + t) * dis + bg_r[...]
  h = jnp.maximum(_layer_norm(u, g_r[...], b_r[...]), 0.0)
  belief = jnp.maximum(
      jnp.dot(h, wref_r[...], preferred_element_type=jnp.float32) + bref_r[...],
      0.0)
  o_r[...] = jnp.dot(belief, wcat_r[...],
                     preferred_element_type=jnp.float32) + bcat_r[...]


def _row_spec(cols):
  return pl.BlockSpec((R, cols), lambda i: (i, 0))


def _full_spec(r, c):
  return pl.BlockSpec((r, c), lambda i: (0, 0))


def _tc_call(body, in_specs, out_cols, args):
  if isinstance(out_cols, tuple):
    out_specs = [_row_spec(cc) for cc in out_cols]
    out_shape = [jax.ShapeDtypeStruct((N, cc), jnp.float32)
                 for cc in out_cols]
  else:
    out_specs = _row_spec(out_cols)
    out_shape = jax.ShapeDtypeStruct((N, out_cols), jnp.float32)
  return pl.pallas_call(
      body,
      grid=(N // R,),
      in_specs=in_specs,
      out_specs=out_specs,
      out_shape=out_shape,
  )(*args)


def kernel(x, edge_index, W_enc, b_enc, W_g1, b_g1, ln1_g, ln1_b, W_g2, b_g2,
           ln2_g, ln2_b, W_ref, b_ref, W_q, b_q, W_f, b_f, W_v, b_v):
  src = edge_index[0].astype(jnp.int32)
  dst = edge_index[1].astype(jnp.int32)
  z16 = jnp.zeros((ZR, HD), jnp.float32)
  z1 = jnp.zeros((RPT, 8), jnp.float32)
  ones1 = jnp.ones((CHUNK, 8), jnp.float32)

  d0, d1 = _sc_degree(dst, z1, ones1)

  t1lo, t1hi, disv = _tc_call(
      _enc_body,
      [_row_spec(12), _row_spec(8), _row_spec(8), _full_spec(12, D),
       _full_spec(1, D), _full_spec(D, D)],
      (HD, HD, 8),
      (x, d0, d1, W_enc, b_enc.reshape(1, D), W_g1))

  S1lo, S1hi = _sc_agg(src, dst, t1lo, t1hi, z16)

  t2lo, t2hi = _tc_call(
      _mid_body,
      [_row_spec(HD), _row_spec(HD), _row_spec(HD), _row_spec(HD),
       _row_spec(8), _full_spec(1, D), _full_spec(1, D),
       _full_spec(1, D), _full_spec(D, D)],
      (HD, HD),
      (S1lo, S1hi, t1lo, t1hi, disv, b_g1.reshape(1, D),
       ln1_g.reshape(1, D), ln1_b.reshape(1, D), W_g2))

  S2lo, S2hi = _sc_agg(src, dst, t2lo, t2hi, z16)

  W_cat = jnp.concatenate([W_q, W_f, W_v], axis=1)
  b_cat = jnp.concatenate([b_q, b_f, b_v]).reshape(1, -1)
  out = _tc_call(
      _out_body,
      [_row_spec(HD), _row_spec(HD), _row_spec(HD), _row_spec(HD),
       _row_spec(8), _full_spec(1, D), _full_spec(1, D),
       _full_spec(1, D), _full_spec(D, D), _full_spec(1, D),
       _full_spec(D, 22), _full_spec(1, 22)],
      22,
      (S2lo, S2hi, t2lo, t2hi, disv, b_g2.reshape(1, D),
       ln2_g.reshape(1, D), ln2_b.reshape(1, D), W_ref, b_ref.reshape(1, D),
       W_cat, b_cat))
  return out
